# baseline (speedup 1.0000x reference)
import functools

import jax
import jax.numpy as jnp
from jax import lax
from jax.experimental import pallas as pl
from jax.experimental.pallas import tpu as pltpu
from jax.experimental.pallas import tpu_sc as plsc

T = 2048
D = 1024
F = 768
E = 64
CAP = 128
S = E * CAP

NC = 2
NS = 16
NW = NC * NS


def _gather_rows_kernel(n_rows, n_cols, chunk):
    per_w = n_rows // NW
    n_ch = per_w // chunk
    mesh = plsc.VectorSubcoreMesh(core_axis_name="c", subcore_axis_name="s")

    @functools.partial(
        pl.kernel,
        out_type=jax.ShapeDtypeStruct((n_rows, n_cols), jnp.float32),
        mesh=mesh,
        scratch_types=[
            pltpu.VMEM((per_w,), jnp.int32),
            pltpu.VMEM((chunk, n_cols), jnp.float32),
            pltpu.VMEM((chunk, n_cols), jnp.float32),
            pltpu.SemaphoreType.DMA,
            pltpu.SemaphoreType.DMA,
        ],
    )
    def gather_k(table_hbm, idx_hbm, out_hbm, idx_v, rows_a, rows_b, sem_a,
                 sem_b):
        wid = lax.axis_index("s") * NC + lax.axis_index("c")
        base = wid * per_w
        pltpu.sync_copy(idx_hbm.at[pl.ds(base, per_w)], idx_v)
        bufs = (rows_a, rows_b)
        sems = (sem_a, sem_b)

        def start(i):
            return pltpu.async_copy(
                table_hbm.at[idx_v.at[pl.ds(i * chunk, chunk)]],
                bufs[i % 2], sems[i % 2])

        cps = [start(0)]
        for i in range(n_ch):
            if i + 1 < n_ch:
                cps.append(start(i + 1))
            cps[i].wait()
            pltpu.sync_copy(bufs[i % 2],
                            out_hbm.at[pl.ds(base + i * chunk, chunk)])

    return gather_k


@functools.cache
def _dispatch_gather():
    return _gather_rows_kernel(S // 2, D, 32)


@functools.cache
def _combine_gather():
    return _gather_rows_kernel(T, D, 32)


EPB = 2
HB = E // (2 * EPB)
NBLK = E // EPB + 1
YROWS = NBLK * EPB * CAP


def _swiglu(x_ref, g_ref, u_ref, d_ref, w_ref, o_ref):
    for j in range(EPB):
        x = x_ref[j]
        g = lax.dot_general(x, g_ref[j], (((1,), (1,)), ((), ())),
                            preferred_element_type=jnp.float32)
        u = lax.dot_general(x, u_ref[j], (((1,), (1,)), ((), ())),
                            preferred_element_type=jnp.float32)
        a = (g * jax.nn.sigmoid(g)) * u
        h = lax.dot_general(a, d_ref[j], (((1,), (1,)), ((), ())),
                            preferred_element_type=jnp.float32)
        o_ref[j] = h * w_ref[j, 0][:, None]


def _mlp_h1_body(x_ref, g_ref, u_ref, d_ref, w_ref, o_ref):
    e = pl.program_id(0)

    @pl.when(e == HB)
    def _zero():
        o_ref[...] = jnp.zeros_like(o_ref)

    @pl.when(e < HB)
    def _compute():
        _swiglu(x_ref, g_ref, u_ref, d_ref, w_ref, o_ref)


def _mlp_h0_body(x_ref, g_ref, u_ref, d_ref, w_ref, yprev_ref, o_ref):
    del yprev_ref
    _swiglu(x_ref, g_ref, u_ref, d_ref, w_ref, o_ref)


def _xmap(e):
    return (jnp.minimum(e, HB - 1), 0, 0)


def _wmap_h1(e):
    return (HB + jnp.minimum(e, HB - 1), 0, 0)


_mlp_h1 = pl.pallas_call(
    _mlp_h1_body,
    grid=(HB + 1,),
    in_specs=[
        pl.BlockSpec((EPB, CAP, D), _xmap),
        pl.BlockSpec((EPB, F, D), _wmap_h1),
        pl.BlockSpec((EPB, F, D), _wmap_h1),
        pl.BlockSpec((EPB, D, F), _wmap_h1),
        pl.BlockSpec((EPB, 1, CAP), _xmap),
    ],
    out_specs=pl.BlockSpec((EPB, CAP, D), lambda e: (HB + e, 0, 0)),
    out_shape=jax.ShapeDtypeStruct((NBLK * EPB, CAP, D), jnp.float32),
    compiler_params=pltpu.CompilerParams(
        dimension_semantics=("arbitrary",)),
)

_mlp_h0 = pl.pallas_call(
    _mlp_h0_body,
    grid=(HB,),
    in_specs=[
        pl.BlockSpec((EPB, CAP, D), lambda e: (e, 0, 0)),
        pl.BlockSpec((EPB, F, D), lambda e: (e, 0, 0)),
        pl.BlockSpec((EPB, F, D), lambda e: (e, 0, 0)),
        pl.BlockSpec((EPB, D, F), lambda e: (e, 0, 0)),
        pl.BlockSpec((EPB, 1, CAP), lambda e: (e, 0, 0)),
        pl.BlockSpec(memory_space=pl.ANY),
    ],
    out_specs=pl.BlockSpec((EPB, CAP, D), lambda e: (e, 0, 0)),
    out_shape=jax.ShapeDtypeStruct((NBLK * EPB, CAP, D), jnp.float32),
    input_output_aliases={5: 0},
    compiler_params=pltpu.CompilerParams(
        dimension_semantics=("arbitrary",)),
)


def kernel(hidden_states, top_k_index, top_k_weights, gate_w, up_w, down_w):
    idx = top_k_index[:, 0].astype(jnp.int32)
    wts = top_k_weights[:, 0]

    G = 16
    GS = T // G
    oh = (idx[:, None] == jnp.arange(E, dtype=jnp.int32)[None, :])
    ohf = oh.astype(jnp.float32)
    ohg = ohf.reshape(G, GS, E)
    r = jnp.arange(GS, dtype=jnp.int32)
    tri = (r[:, None] >= r[None, :]).astype(jnp.float32)
    within = jnp.einsum('ij,gje->gie', tri, ohg,
                        preferred_element_type=jnp.float32)
    gsum = within[:, -1, :]
    offs = jnp.cumsum(gsum, axis=0) - gsum
    pos = (within + offs[:, None, :]).reshape(T, E)
    p = (jnp.sum(pos * ohf, axis=1) - 1.0).astype(jnp.int32)
    keep = p < CAP
    slot = jnp.where(keep, idx * CAP + p, S)

    arange_t = jnp.arange(T, dtype=jnp.int32)
    fill = jnp.arange(S + 1, dtype=jnp.int32) % T
    tok = fill.at[slot].set(arange_t)[:S]
    w_all = jnp.zeros((S + 1,), jnp.float32).at[slot].set(wts)[:S]

    w3 = w_all.reshape(E, 1, CAP)
    x1 = _dispatch_gather()(hidden_states, tok[S // 2:])
    y_part = _mlp_h1(x1.reshape(E // 2, CAP, D), gate_w, up_w, down_w,
                     w3[E // 2:])
    x0 = _dispatch_gather()(hidden_states, tok[:S // 2])
    y = _mlp_h0(x0.reshape(E // 2, CAP, D), gate_w, up_w, down_w,
                w3[:E // 2], y_part)
    out = _combine_gather()(y.reshape(YROWS, D), slot)
    return out


# speedup vs baseline: 1.0294x; 1.0294x over previous
import functools

import jax
import jax.numpy as jnp
from jax import lax
from jax.experimental import pallas as pl
from jax.experimental.pallas import tpu as pltpu
from jax.experimental.pallas import tpu_sc as plsc

T = 2048
D = 1024
F = 768
E = 64
CAP = 128
S = E * CAP

NC = 2
NS = 16
NW = NC * NS


def _gather_rows_kernel(n_rows, n_cols, chunk):
    per_w = n_rows // NW
    n_ch = per_w // chunk
    mesh = plsc.VectorSubcoreMesh(core_axis_name="c", subcore_axis_name="s")

    @functools.partial(
        pl.kernel,
        out_type=jax.ShapeDtypeStruct((n_rows, n_cols), jnp.float32),
        mesh=mesh,
        scratch_types=[
            pltpu.VMEM((per_w,), jnp.int32),
            pltpu.VMEM((chunk, n_cols), jnp.float32),
            pltpu.VMEM((chunk, n_cols), jnp.float32),
            pltpu.SemaphoreType.DMA,
            pltpu.SemaphoreType.DMA,
        ],
    )
    def gather_k(table_hbm, idx_hbm, out_hbm, idx_v, rows_a, rows_b, sem_a,
                 sem_b):
        wid = lax.axis_index("s") * NC + lax.axis_index("c")
        base = wid * per_w
        pltpu.sync_copy(idx_hbm.at[pl.ds(base, per_w)], idx_v)
        bufs = (rows_a, rows_b)
        sems = (sem_a, sem_b)

        def start(i):
            return pltpu.async_copy(
                table_hbm.at[idx_v.at[pl.ds(i * chunk, chunk)]],
                bufs[i % 2], sems[i % 2])

        cps = [start(0)]
        for i in range(n_ch):
            if i + 1 < n_ch:
                cps.append(start(i + 1))
            cps[i].wait()
            pltpu.sync_copy(bufs[i % 2],
                            out_hbm.at[pl.ds(base + i * chunk, chunk)])

    return gather_k


@functools.cache
def _combine_gather():
    return _gather_rows_kernel(T, D, 32)


YROWS = (E + 1) * CAP


def _mlp_body(hs_ref, g_ref, u_ref, d_ref, w_ref, tok_ref, o_ref):
    e = pl.program_id(0)

    @pl.when(e == E)
    def _zero():
        o_ref[...] = jnp.zeros_like(o_ref)

    @pl.when(e < E)
    def _compute():
        pt = (lax.broadcasted_iota(jnp.int32, (T, CAP), 0)
              == tok_ref[0]).astype(jnp.float32)
        x = lax.dot_general(pt, hs_ref[...], (((0,), (0,)), ((), ())),
                            preferred_element_type=jnp.float32)
        g = lax.dot_general(x, g_ref[0], (((1,), (1,)), ((), ())),
                            preferred_element_type=jnp.float32)
        u = lax.dot_general(x, u_ref[0], (((1,), (1,)), ((), ())),
                            preferred_element_type=jnp.float32)
        a = (g * jax.nn.sigmoid(g)) * u
        h = lax.dot_general(a, d_ref[0], (((1,), (1,)), ((), ())),
                            preferred_element_type=jnp.float32)
        o_ref[0] = h * w_ref[0, 0][:, None]


def _wmap(e):
    return (jnp.minimum(e, E - 1), 0, 0)


_mlp_call = pl.pallas_call(
    _mlp_body,
    grid=(E + 1,),
    in_specs=[
        pl.BlockSpec((T, D), lambda e: (0, 0)),
        pl.BlockSpec((1, F, D), _wmap),
        pl.BlockSpec((1, F, D), _wmap),
        pl.BlockSpec((1, D, F), _wmap),
        pl.BlockSpec((1, 1, CAP), _wmap),
        pl.BlockSpec((1, 1, CAP), _wmap),
    ],
    out_specs=pl.BlockSpec((1, CAP, D), lambda e: (e, 0, 0)),
    out_shape=jax.ShapeDtypeStruct((E + 1, CAP, D), jnp.float32),
    compiler_params=pltpu.CompilerParams(
        dimension_semantics=("arbitrary",)),
)


def kernel(hidden_states, top_k_index, top_k_weights, gate_w, up_w, down_w):
    idx = top_k_index[:, 0].astype(jnp.int32)
    wts = top_k_weights[:, 0]

    G = 16
    GS = T // G
    oh = (idx[:, None] == jnp.arange(E, dtype=jnp.int32)[None, :])
    ohf = oh.astype(jnp.float32)
    ohg = ohf.reshape(G, GS, E)
    r = jnp.arange(GS, dtype=jnp.int32)
    tri = (r[:, None] >= r[None, :]).astype(jnp.float32)
    within = jnp.einsum('ij,gje->gie', tri, ohg,
                        preferred_element_type=jnp.float32)
    gsum = within[:, -1, :]
    offs = jnp.cumsum(gsum, axis=0) - gsum
    pos = (within + offs[:, None, :]).reshape(T, E)
    p = (jnp.sum(pos * ohf, axis=1) - 1.0).astype(jnp.int32)
    keep = p < CAP
    slot = jnp.where(keep, idx * CAP + p, S)

    arange_t = jnp.arange(T, dtype=jnp.int32)
    fill = jnp.arange(S + 1, dtype=jnp.int32) % T
    tok = fill.at[slot].set(arange_t)[:S]
    w_all = jnp.zeros((S + 1,), jnp.float32).at[slot].set(wts)[:S]

    y = _mlp_call(hidden_states, gate_w, up_w, down_w,
                  w_all.reshape(E, 1, CAP), tok.reshape(E, 1, CAP))
    out = _combine_gather()(y.reshape(YROWS, D), slot)
    return out


# speedup vs baseline: 1.0368x; 1.0071x over previous
import functools

import jax
import jax.numpy as jnp
from jax import lax
from jax.experimental import pallas as pl
from jax.experimental.pallas import tpu as pltpu
from jax.experimental.pallas import tpu_sc as plsc

T = 2048
D = 1024
F = 768
E = 64
CAP = 128
S = E * CAP

NC = 2
NS = 16
NW = NC * NS


def _gather_rows_kernel(n_rows, n_cols, chunk):
    per_w = n_rows // NW
    n_ch = per_w // chunk
    mesh = plsc.VectorSubcoreMesh(core_axis_name="c", subcore_axis_name="s")

    @functools.partial(
        pl.kernel,
        out_type=jax.ShapeDtypeStruct((n_rows, n_cols), jnp.float32),
        mesh=mesh,
        scratch_types=[
            pltpu.VMEM((per_w,), jnp.int32),
            pltpu.VMEM((chunk, n_cols), jnp.float32),
            pltpu.VMEM((chunk, n_cols), jnp.float32),
            pltpu.SemaphoreType.DMA,
            pltpu.SemaphoreType.DMA,
        ],
    )
    def gather_k(table_hbm, idx_hbm, out_hbm, idx_v, rows_a, rows_b, sem_a,
                 sem_b):
        wid = lax.axis_index("s") * NC + lax.axis_index("c")
        base = wid * per_w
        pltpu.sync_copy(idx_hbm.at[pl.ds(base, per_w)], idx_v)
        bufs = (rows_a, rows_b)
        sems = (sem_a, sem_b)

        def start(i):
            return pltpu.async_copy(
                table_hbm.at[idx_v.at[pl.ds(i * chunk, chunk)]],
                bufs[i % 2], sems[i % 2])

        cps = [start(0)]
        for i in range(n_ch):
            if i + 1 < n_ch:
                cps.append(start(i + 1))
            cps[i].wait()
            pltpu.sync_copy(bufs[i % 2],
                            out_hbm.at[pl.ds(base + i * chunk, chunk)])

    return gather_k


@functools.cache
def _combine_gather():
    return _gather_rows_kernel(T, D, 32)


YROWS = (E + 1) * CAP


def _mlp_body(hs_ref, ga_ref, gb_ref, ua_ref, ub_ref, da_ref, db_ref,
              w_ref, tok_ref, o_ref):
    e = pl.program_id(0)

    @pl.when(e == E)
    def _zero():
        o_ref[...] = jnp.zeros_like(o_ref)

    @pl.when(e < E)
    def _compute():
        pt = (lax.broadcasted_iota(jnp.int32, (T, CAP), 0)
              == tok_ref[0]).astype(jnp.float32)
        x = lax.dot_general(pt, hs_ref[...], (((0,), (0,)), ((), ())),
                            preferred_element_type=jnp.float32)
        h = None
        for gh_ref, uh_ref, dh_ref in ((ga_ref, ua_ref, da_ref),
                                       (gb_ref, ub_ref, db_ref)):
            g = lax.dot_general(x, gh_ref[0], (((1,), (1,)), ((), ())),
                                preferred_element_type=jnp.float32)
            u = lax.dot_general(x, uh_ref[0], (((1,), (1,)), ((), ())),
                                preferred_element_type=jnp.float32)
            a = (g * jax.nn.sigmoid(g)) * u
            hh = lax.dot_general(a, dh_ref[0], (((1,), (1,)), ((), ())),
                                 preferred_element_type=jnp.float32)
            h = hh if h is None else h + hh
        o_ref[0] = h * w_ref[0, 0][:, None]


def _wmap(e):
    return (jnp.minimum(e, E - 1), 0, 0)


def _wmap_b(e):
    return (jnp.minimum(e, E - 1), 1, 0)


def _wmap_db(e):
    return (jnp.minimum(e, E - 1), 0, 1)


_mlp_call = pl.pallas_call(
    _mlp_body,
    grid=(E + 1,),
    in_specs=[
        pl.BlockSpec((T, D), lambda e: (0, 0)),
        pl.BlockSpec((1, F // 2, D), _wmap),
        pl.BlockSpec((1, F // 2, D), _wmap_b),
        pl.BlockSpec((1, F // 2, D), _wmap),
        pl.BlockSpec((1, F // 2, D), _wmap_b),
        pl.BlockSpec((1, D, F // 2), _wmap),
        pl.BlockSpec((1, D, F // 2), _wmap_db),
        pl.BlockSpec((1, 1, CAP), _wmap),
        pl.BlockSpec((1, 1, CAP), _wmap),
    ],
    out_specs=pl.BlockSpec((1, CAP, D), lambda e: (e, 0, 0)),
    out_shape=jax.ShapeDtypeStruct((E + 1, CAP, D), jnp.float32),
    compiler_params=pltpu.CompilerParams(
        dimension_semantics=("arbitrary",)),
)


def kernel(hidden_states, top_k_index, top_k_weights, gate_w, up_w, down_w):
    idx = top_k_index[:, 0].astype(jnp.int32)
    wts = top_k_weights[:, 0]

    G = 16
    GS = T // G
    oh = (idx[:, None] == jnp.arange(E, dtype=jnp.int32)[None, :])
    ohf = oh.astype(jnp.float32)
    ohg = ohf.reshape(G, GS, E)
    r = jnp.arange(GS, dtype=jnp.int32)
    tri = (r[:, None] >= r[None, :]).astype(jnp.float32)
    within = jnp.einsum('ij,gje->gie', tri, ohg,
                        preferred_element_type=jnp.float32)
    gsum = within[:, -1, :]
    offs = jnp.cumsum(gsum, axis=0) - gsum
    pos = (within + offs[:, None, :]).reshape(T, E)
    p = (jnp.sum(pos * ohf, axis=1) - 1.0).astype(jnp.int32)
    keep = p < CAP
    slot = jnp.where(keep, idx * CAP + p, S)

    arange_t = jnp.arange(T, dtype=jnp.int32)
    fill = jnp.arange(S + 1, dtype=jnp.int32) % T
    tok = fill.at[slot].set(arange_t)[:S]
    w_all = jnp.zeros((S + 1,), jnp.float32).at[slot].set(wts)[:S]

    y = _mlp_call(hidden_states, gate_w, gate_w, up_w, up_w, down_w, down_w,
                  w_all.reshape(E, 1, CAP), tok.reshape(E, 1, CAP))
    out = _combine_gather()(y.reshape(YROWS, D), slot)
    return out


# speedup vs baseline: 1.1124x; 1.0729x over previous
import functools

import jax
import jax.numpy as jnp
from jax import lax
from jax.experimental import pallas as pl
from jax.experimental.pallas import tpu as pltpu
from jax.experimental.pallas import tpu_sc as plsc

T = 2048
D = 1024
F = 768
E = 64
CAP = 128
S = E * CAP

NC = 2
NS = 16
NW = NC * NS


def _gather_rows_kernel(n_rows, n_cols, chunk):
    per_w = n_rows // NW
    n_ch = per_w // chunk
    mesh = plsc.VectorSubcoreMesh(core_axis_name="c", subcore_axis_name="s")

    @functools.partial(
        pl.kernel,
        out_type=jax.ShapeDtypeStruct((n_rows, n_cols), jnp.float32),
        mesh=mesh,
        scratch_types=[
            pltpu.VMEM((per_w,), jnp.int32),
            pltpu.VMEM((chunk, n_cols), jnp.float32),
            pltpu.VMEM((chunk, n_cols), jnp.float32),
            pltpu.SemaphoreType.DMA,
            pltpu.SemaphoreType.DMA,
        ],
    )
    def gather_k(table_hbm, idx_hbm, out_hbm, idx_v, rows_a, rows_b, sem_a,
                 sem_b):
        wid = lax.axis_index("s") * NC + lax.axis_index("c")
        base = wid * per_w
        pltpu.sync_copy(idx_hbm.at[pl.ds(base, per_w)], idx_v)
        bufs = (rows_a, rows_b)
        sems = (sem_a, sem_b)

        def start(i):
            return pltpu.async_copy(
                table_hbm.at[idx_v.at[pl.ds(i * chunk, chunk)]],
                bufs[i % 2], sems[i % 2])

        cps = [start(0)]
        for i in range(n_ch):
            if i + 1 < n_ch:
                cps.append(start(i + 1))
            cps[i].wait()
            pltpu.sync_copy(bufs[i % 2],
                            out_hbm.at[pl.ds(base + i * chunk, chunk)])

    return gather_k


@functools.cache
def _combine_gather():
    return _gather_rows_kernel(T, D, 32)


EPB = 2
NST = E // EPB + 1
YROWS = NST * EPB * CAP


def _mlp_body(hs_ref, ga_ref, gb_ref, ua_ref, ub_ref, da_ref, db_ref,
              w_ref, tok_ref, o_ref):
    e = pl.program_id(0)

    @pl.when(e == NST - 1)
    def _zero():
        o_ref[...] = jnp.zeros_like(o_ref)

    @pl.when(e < NST - 1)
    def _compute():
        for j in range(EPB):
            pt = (lax.broadcasted_iota(jnp.int32, (T, CAP), 0)
                  == tok_ref[j]).astype(jnp.float32)
            x = lax.dot_general(pt, hs_ref[...], (((0,), (0,)), ((), ())),
                                preferred_element_type=jnp.float32)
            h = None
            for gh_ref, uh_ref, dh_ref in ((ga_ref, ua_ref, da_ref),
                                           (gb_ref, ub_ref, db_ref)):
                g = lax.dot_general(x, gh_ref[j], (((1,), (1,)), ((), ())),
                                    preferred_element_type=jnp.float32)
                u = lax.dot_general(x, uh_ref[j], (((1,), (1,)), ((), ())),
                                    preferred_element_type=jnp.float32)
                a = (g * jax.nn.sigmoid(g)) * u
                hh = lax.dot_general(a, dh_ref[j], (((1,), (1,)), ((), ())),
                                     preferred_element_type=jnp.float32)
                h = hh if h is None else h + hh
            o_ref[j] = h * w_ref[j, 0][:, None]


def _wmap(e):
    return (jnp.minimum(e, E // EPB - 1), 0, 0)


def _wmap_b(e):
    return (jnp.minimum(e, E // EPB - 1), 1, 0)


def _wmap_db(e):
    return (jnp.minimum(e, E // EPB - 1), 0, 1)


_mlp_call = pl.pallas_call(
    _mlp_body,
    grid=(NST,),
    in_specs=[
        pl.BlockSpec((T, D), lambda e: (0, 0)),
        pl.BlockSpec((EPB, F // 2, D), _wmap),
        pl.BlockSpec((EPB, F // 2, D), _wmap_b),
        pl.BlockSpec((EPB, F // 2, D), _wmap),
        pl.BlockSpec((EPB, F // 2, D), _wmap_b),
        pl.BlockSpec((EPB, D, F // 2), _wmap),
        pl.BlockSpec((EPB, D, F // 2), _wmap_db),
        pl.BlockSpec((EPB, 1, CAP), _wmap),
        pl.BlockSpec((EPB, 1, CAP), _wmap),
    ],
    out_specs=pl.BlockSpec((EPB, CAP, D), lambda e: (e, 0, 0)),
    out_shape=jax.ShapeDtypeStruct((NST * EPB, CAP, D), jnp.float32),
    compiler_params=pltpu.CompilerParams(
        dimension_semantics=("arbitrary",)),
)


def kernel(hidden_states, top_k_index, top_k_weights, gate_w, up_w, down_w):
    idx = top_k_index[:, 0].astype(jnp.int32)
    wts = top_k_weights[:, 0]

    G = 16
    GS = T // G
    oh = (idx[:, None] == jnp.arange(E, dtype=jnp.int32)[None, :])
    ohf = oh.astype(jnp.float32)
    ohg = ohf.reshape(G, GS, E)
    r = jnp.arange(GS, dtype=jnp.int32)
    tri = (r[:, None] >= r[None, :]).astype(jnp.float32)
    within = jnp.einsum('ij,gje->gie', tri, ohg,
                        preferred_element_type=jnp.float32)
    gsum = within[:, -1, :]
    offs = jnp.cumsum(gsum, axis=0) - gsum
    pos = (within + offs[:, None, :]).reshape(T, E)
    p = (jnp.sum(pos * ohf, axis=1) - 1.0).astype(jnp.int32)
    keep = p < CAP
    slot = jnp.where(keep, idx * CAP + p, S)

    arange_t = jnp.arange(T, dtype=jnp.int32)
    fill = jnp.arange(S + 1, dtype=jnp.int32) % T
    tok = fill.at[slot].set(arange_t)[:S]
    w_all = jnp.zeros((S + 1,), jnp.float32).at[slot].set(wts)[:S]

    y = _mlp_call(hidden_states, gate_w, gate_w, up_w, up_w, down_w, down_w,
                  w_all.reshape(E, 1, CAP), tok.reshape(E, 1, CAP))
    out = _combine_gather()(y.reshape(YROWS, D), slot)
    return out
